# baseline (speedup 1.0000x reference)
import jax
import jax.numpy as jnp
from jax import lax
from jax.experimental import pallas as pl
from jax.experimental.pallas import tpu as pltpu
from jax.experimental.pallas import tpu_sc as plsc

_WINDOW = 512
_TBLK = 16384
_BS = 2048


def _lane_fold_mats(dim, dtype):
    d_io = lax.broadcasted_iota(jnp.int32, (dim, 8 * dim), 0)
    c_io = lax.broadcasted_iota(jnp.int32, (dim, 8 * dim), 1)
    return [(c_io == dim * k + d_io).astype(dtype) for k in range(8)]


def _pack_table(tab_t):
    dim, vocab = tab_t.shape
    grid = (vocab + _TBLK - 1) // _TBLK

    def body(x_ref, o_ref):
        xt = x_ref[...].T
        xt3 = xt.reshape(_TBLK // 8, 8, dim)
        mats = _lane_fold_mats(dim, xt.dtype)
        acc = jnp.dot(xt3[:, 0, :], mats[0], preferred_element_type=jnp.float32)
        for k in range(1, 8):
            acc += jnp.dot(xt3[:, k, :], mats[k], preferred_element_type=jnp.float32)
        o_ref[...] = acc

    return pl.pallas_call(
        body,
        grid=(grid,),
        in_specs=[pl.BlockSpec((dim, _TBLK), lambda i: (0, i))],
        out_specs=pl.BlockSpec((_TBLK * dim // 128, 128), lambda i: (i, 0)),
        out_shape=jax.ShapeDtypeStruct((vocab * dim // 128, 128), tab_t.dtype),
    )(tab_t)


def _sc_gather(tab_lin, idx_lin, n, dim):
    mesh = plsc.VectorSubcoreMesh(core_axis_name="c", subcore_axis_name="s")

    @pl.kernel(
        out_type=jax.ShapeDtypeStruct((n, dim), tab_lin.dtype),
        mesh=mesh,
        compiler_params=pltpu.CompilerParams(use_tc_tiling_on_sc=False),
    )
    def gather_kernel(tab_hbm, idx_hbm, out_hbm):
        def body(idx_v, out_v):
            pltpu.sync_copy(tab_hbm.at[idx_v.at[0]], out_v)

        pltpu.emit_pipeline(
            body,
            grid=(n // _WINDOW,),
            in_specs=[pl.BlockSpec((1, _WINDOW), index_map=lambda i: (0, i))],
            out_specs=[pl.BlockSpec((_WINDOW, dim), index_map=lambda i: (i, 0))],
            core_axis_name=("c", "s"),
            dimension_semantics=(pltpu.PARALLEL,),
        )(idx_hbm, out_hbm)

    return gather_kernel(tab_lin, idx_lin)


def _unpack_out(g2, hist, batch, dim):
    rows_per_bs = _BS * dim // 128

    def body(x_ref, o_ref):
        x = x_ref[...]
        cols = [x[:, dim * k:dim * (k + 1)] for k in range(8)]
        st = jnp.stack(cols, axis=2)
        o_ref[0] = st.transpose(1, 0, 2).reshape(dim, _BS)

    return pl.pallas_call(
        body,
        grid=(hist, batch // _BS),
        in_specs=[pl.BlockSpec((rows_per_bs, 128), lambda h, c: (h * (batch // _BS) + c, 0))],
        out_specs=pl.BlockSpec((1, dim, _BS), lambda h, c: (h, 0, c)),
        out_shape=jax.ShapeDtypeStruct((hist, dim, batch), g2.dtype),
    )(g2)


def kernel(input_idx, table):
    batch, hist = input_idx.shape
    vocab, dim = table.shape
    n = batch * hist

    tab_lin = _pack_table(table.T).reshape(vocab, dim)
    idx_lin = input_idx.T.astype(jnp.int32).reshape(1, n)
    g = _sc_gather(tab_lin, idx_lin, n, dim)
    out3t = _unpack_out(g.reshape(n * dim // 128, 128), hist, batch, dim)
    return out3t.transpose(2, 0, 1)


# speedup vs baseline: 3.8010x; 3.8010x over previous
import jax
import jax.numpy as jnp
from jax import lax
from jax.experimental import pallas as pl
from jax.experimental.pallas import tpu as pltpu
from jax.experimental.pallas import tpu_sc as plsc

_WINDOW = 512
_TBLK = 16384
_BS = 2048


def _lane_fold_mats(dim, dtype):
    d_io = lax.broadcasted_iota(jnp.int32, (dim, 8 * dim), 0)
    c_io = lax.broadcasted_iota(jnp.int32, (dim, 8 * dim), 1)
    return [(c_io == dim * k + d_io).astype(dtype) for k in range(8)]


def _pack_table(tab_t):
    dim, vocab = tab_t.shape
    grid = (vocab + _TBLK - 1) // _TBLK

    def body(x_ref, o_ref):
        xt = x_ref[...].T
        xt3 = xt.reshape(_TBLK // 8, 8, dim)
        mats = _lane_fold_mats(dim, xt.dtype)
        acc = jnp.dot(xt3[:, 0, :], mats[0], preferred_element_type=jnp.float32)
        for k in range(1, 8):
            acc += jnp.dot(xt3[:, k, :], mats[k], preferred_element_type=jnp.float32)
        o_ref[...] = acc

    return pl.pallas_call(
        body,
        grid=(grid,),
        in_specs=[pl.BlockSpec((dim, _TBLK), lambda i: (0, i))],
        out_specs=pl.BlockSpec((_TBLK * dim // 128, 128), lambda i: (i, 0)),
        out_shape=jax.ShapeDtypeStruct((vocab * dim // 128, 128), tab_t.dtype),
    )(tab_t)


def _sc_gather(tab_lin, idx_lin, n, dim):
    mesh = plsc.VectorSubcoreMesh(core_axis_name="c", subcore_axis_name="s")

    @pl.kernel(
        out_type=jax.ShapeDtypeStruct((n, dim), tab_lin.dtype),
        mesh=mesh,
        compiler_params=pltpu.CompilerParams(use_tc_tiling_on_sc=False),
    )
    def gather_kernel(tab_hbm, idx_hbm, out_hbm):
        def body(idx_v, out_v):
            pltpu.sync_copy(tab_hbm.at[idx_v.at[0]], out_v)

        pltpu.emit_pipeline(
            body,
            grid=(n // _WINDOW,),
            in_specs=[pl.BlockSpec((1, _WINDOW), index_map=lambda i: (0, i))],
            out_specs=[pl.BlockSpec((_WINDOW, dim), index_map=lambda i: (i, 0))],
            core_axis_name=("c", "s"),
            dimension_semantics=(pltpu.PARALLEL,),
        )(idx_hbm, out_hbm)

    return gather_kernel(tab_lin, idx_lin)


def _unpack_out(g2, hist, batch, dim):
    rows_per_bs = _BS * dim // 128

    def body(x_ref, o_ref):
        x = x_ref[...]
        cols = [x[:, dim * k:dim * (k + 1)] for k in range(8)]
        st = jnp.stack(cols, axis=2)
        o_ref[0] = st.transpose(1, 0, 2).reshape(dim, _BS)

    return pl.pallas_call(
        body,
        grid=(hist, batch // _BS),
        in_specs=[pl.BlockSpec((rows_per_bs, 128), lambda h, c: (h * (batch // _BS) + c, 0))],
        out_specs=pl.BlockSpec((1, dim, _BS), lambda h, c: (h, 0, c)),
        out_shape=jax.ShapeDtypeStruct((hist, dim, batch), g2.dtype),
    )(g2)


def kernel(input_idx, table):
    batch, hist = input_idx.shape
    vocab, dim = table.shape
    n = batch * hist

    tab_lin = _pack_table(table.T).reshape(vocab, dim)
    idx_lin = input_idx.T.astype(jnp.int32).reshape(1, n)
    g = _sc_gather(tab_lin, idx_lin, n, dim)
    return g.reshape(hist, batch, dim).transpose(1, 0, 2)
